# baseline (speedup 1.0000x reference)
import functools

import jax
import jax.numpy as jnp
from jax import lax
from jax.experimental import pallas as pl
from jax.experimental.pallas import tpu as pltpu
from jax.experimental.pallas import tpu_sc as plsc

BATCH = 16384
EMBED = 16
N_ROWS = 1000000
REG_SCALE = 1e-05 * 0.5

_INFO = plsc.get_sparse_core_info()
NC = _INFO.num_cores
NS = _INFO.num_subcores
NW = NC * NS
BPW = BATCH // NW
GROUPS = BPW // 16


CHUNK = 128
NCHUNK = BPW // CHUNK
CGROUPS = CHUNK // 16


def _sc_body(user_h, pos_h, neg_h, ue_h, ie_h,
             d_out, acc_out,
             idx_u, idx_p, idx_n,
             blk_u, blk_p, blk_n,
             bu_v, bp_v, bn_v, d_v, acc_v, sem):
    wid = lax.axis_index("s") * NC + lax.axis_index("c")
    base = wid * BPW

    pltpu.sync_copy(user_h.at[pl.ds(base, BPW)], idx_u)
    pltpu.sync_copy(pos_h.at[pl.ds(base, BPW)], idx_p)
    pltpu.sync_copy(neg_h.at[pl.ds(base, BPW)], idx_n)

    def mkblk(s, idx_ref, blk_ref):
        v = idx_ref[pl.ds(s * 16, 16)]
        blk_ref[pl.ds(s * 16, 16)] = lax.shift_right_logical(v, 3)

    @pl.loop(0, BPW // 16)
    def _(s):
        mkblk(s, idx_u, blk_u)
        mkblk(s, idx_p, blk_p)
        mkblk(s, idx_n, blk_n)

    row0 = lax.iota(jnp.int32, 16)

    def chunk_gather(c):
        cb = c * CHUNK
        cu = pltpu.async_copy(ue_h.at[blk_u.at[pl.ds(cb, CHUNK)]], bu_v, sem)
        cp = pltpu.async_copy(ie_h.at[blk_p.at[pl.ds(cb, CHUNK)]], bp_v, sem)
        cn = pltpu.async_copy(ie_h.at[blk_n.at[pl.ds(cb, CHUNK)]], bn_v, sem)
        return cu, cp, cn

    def chunk_compute(c, acc):
        cb = c * CHUNK

        def group(g, acc):
            rows = g * 16 + row0
            off_u = (idx_u[pl.ds(cb + g * 16, 16)] & 7) * 16
            off_p = (idx_p[pl.ds(cb + g * 16, 16)] & 7) * 16
            off_n = (idx_n[pl.ds(cb + g * 16, 16)] & 7) * 16
            dvec = jnp.zeros((16,), jnp.float32)
            for j in range(16):
                uc = plsc.load_gather(bu_v, [rows, off_u + j])
                pc = plsc.load_gather(bp_v, [rows, off_p + j])
                nc = plsc.load_gather(bn_v, [rows, off_n + j])
                dvec = dvec + uc * (pc - nc)
                acc = acc + uc * uc + pc * pc + nc * nc
            d_v[pl.ds(cb + g * 16, 16)] = dvec
            return acc

        return lax.fori_loop(0, CGROUPS, group, acc)

    def run_chunk(c, acc):
        cu, cp, cn = chunk_gather(c)
        cu.wait()
        cp.wait()
        cn.wait()
        return chunk_compute(c, acc)

    acc = lax.fori_loop(0, NCHUNK, run_chunk, jnp.zeros((16,), jnp.float32))
    acc_v[...] = acc

    pltpu.sync_copy(d_v, d_out.at[pl.ds(base, BPW)])
    pltpu.sync_copy(acc_v, acc_out.at[wid])


@functools.partial(
    pl.kernel,
    mesh=plsc.VectorSubcoreMesh(core_axis_name="c", subcore_axis_name="s"),
    compiler_params=pltpu.CompilerParams(
        needs_layout_passes=False, use_tc_tiling_on_sc=False),
    out_type=[
        jax.ShapeDtypeStruct((BATCH,), jnp.float32),
        jax.ShapeDtypeStruct((NW, EMBED), jnp.float32),
    ],
    scratch_types=[
        pltpu.VMEM((BPW,), jnp.int32),
        pltpu.VMEM((BPW,), jnp.int32),
        pltpu.VMEM((BPW,), jnp.int32),
        pltpu.VMEM((BPW,), jnp.int32),
        pltpu.VMEM((BPW,), jnp.int32),
        pltpu.VMEM((BPW,), jnp.int32),
        pltpu.VMEM((CHUNK, 128), jnp.float32),
        pltpu.VMEM((CHUNK, 128), jnp.float32),
        pltpu.VMEM((CHUNK, 128), jnp.float32),
        pltpu.VMEM((BPW,), jnp.float32),
        pltpu.VMEM((EMBED,), jnp.float32),
        pltpu.SemaphoreType.DMA,
    ],
)
def _sc_kernel(user_h, pos_h, neg_h, ue_h, ie_h, d_out, acc_out,
               idx_u, idx_p, idx_n, blk_u, blk_p, blk_n,
               bu_v, bp_v, bn_v, d_v, acc_v, sem):
    _sc_body(user_h, pos_h, neg_h, ue_h, ie_h, d_out, acc_out,
             idx_u, idx_p, idx_n, blk_u, blk_p, blk_n,
             bu_v, bp_v, bn_v, d_v, acc_v, sem)


def _tc_body(d_ref, acc_ref, bpr_ref, reg_ref):
    x = d_ref[...]
    s = jnp.log(jax.nn.sigmoid(x))
    bpr_ref[0, 0] = -jnp.sum(s) / jnp.float32(BATCH)
    reg_ref[0, 0] = jnp.float32(REG_SCALE) * jnp.sum(acc_ref[...])


_tc_finish = pl.pallas_call(
    _tc_body,
    out_shape=[
        jax.ShapeDtypeStruct((1, 1), jnp.float32),
        jax.ShapeDtypeStruct((1, 1), jnp.float32),
    ],
    in_specs=[
        pl.BlockSpec(memory_space=pltpu.VMEM),
        pl.BlockSpec(memory_space=pltpu.VMEM),
    ],
    out_specs=[
        pl.BlockSpec(memory_space=pltpu.SMEM),
        pl.BlockSpec(memory_space=pltpu.SMEM),
    ],
)


def kernel(user, pos, neg, user_embedding, item_embedding):
    user = user.astype(jnp.int32)
    pos = pos.astype(jnp.int32)
    neg = neg.astype(jnp.int32)
    d, acc = _sc_kernel(user, pos, neg,
                        user_embedding.reshape(N_ROWS // 8, 128),
                        item_embedding.reshape(N_ROWS // 8, 128))
    bpr, reg = _tc_finish(d.reshape(128, 128), acc)
    return (bpr[0, 0], reg[0, 0])


# speedup vs baseline: 1.5115x; 1.5115x over previous
import functools

import jax
import jax.numpy as jnp
from jax import lax
from jax.experimental import pallas as pl
from jax.experimental.pallas import tpu as pltpu
from jax.experimental.pallas import tpu_sc as plsc

BATCH = 16384
EMBED = 16
N_ROWS = 1000000
REG_SCALE = 1e-05 * 0.5

_INFO = plsc.get_sparse_core_info()
NC = _INFO.num_cores
NS = _INFO.num_subcores
NW = NC * NS
BPW = BATCH // NW
GROUPS = BPW // 16
DEPTH = 4


def _issue_group(s, idx_u, idx_p, idx_n, ue_h, ie_h, bufs, sem):
    slot = lax.rem(s, DEPTH)
    iv_u = idx_u[pl.ds(s * 16, 16)]
    iv_p = idx_p[pl.ds(s * 16, 16)]
    iv_n = idx_n[pl.ds(s * 16, 16)]
    bu, bp, bn = bufs
    for k in range(16):
        ru = jnp.squeeze(lax.slice(iv_u, (k,), (k + 1,)))
        rp = jnp.squeeze(lax.slice(iv_p, (k,), (k + 1,)))
        rn = jnp.squeeze(lax.slice(iv_n, (k,), (k + 1,)))
        pltpu.async_copy(ue_h.at[pl.ds(ru, 1)], bu.at[slot, pl.ds(k, 1)], sem)
        pltpu.async_copy(ie_h.at[pl.ds(rp, 1)], bp.at[slot, pl.ds(k, 1)], sem)
        pltpu.async_copy(ie_h.at[pl.ds(rn, 1)], bn.at[slot, pl.ds(k, 1)], sem)


def _drain_group(s, ue_h, bufs, sem):
    slot = lax.rem(s, DEPTH)
    bu, bp, bn = bufs
    for k in range(16):
        pltpu.make_async_copy(ue_h.at[pl.ds(0, 1)], bu.at[slot, pl.ds(k, 1)], sem).wait()
        pltpu.make_async_copy(ue_h.at[pl.ds(0, 1)], bp.at[slot, pl.ds(k, 1)], sem).wait()
        pltpu.make_async_copy(ue_h.at[pl.ds(0, 1)], bn.at[slot, pl.ds(k, 1)], sem).wait()


def _compute_group(s, acc, bufs, d_v, lane):
    slot = lax.rem(s, DEPTH)
    bu, bp, bn = bufs
    dvec = jnp.zeros((16,), jnp.float32)
    for k in range(16):
        u = bu[slot, k]
        p = bp[slot, k]
        n = bn[slot, k]
        q = u * (p - n)
        acc = acc + u * u + p * p + n * n
        dvec = jnp.where(lane == k, jnp.sum(q), dvec)
    d_v[pl.ds(s * 16, 16)] = dvec
    return acc


def _sc_body(user_h, pos_h, neg_h, ue_h, ie_h,
             d_out, acc_out,
             idx_u, idx_p, idx_n,
             bu_v, bp_v, bn_v, d_v, acc_v, sem):
    wid = lax.axis_index("s") * NC + lax.axis_index("c")
    base = wid * BPW

    pltpu.sync_copy(user_h.at[pl.ds(base, BPW)], idx_u)
    pltpu.sync_copy(pos_h.at[pl.ds(base, BPW)], idx_p)
    pltpu.sync_copy(neg_h.at[pl.ds(base, BPW)], idx_n)

    lane = lax.iota(jnp.int32, 16)
    bufs = (bu_v, bp_v, bn_v)

    for s in range(DEPTH):
        _issue_group(jnp.int32(s), idx_u, idx_p, idx_n, ue_h, ie_h, bufs, sem)

    def step(s, acc):
        _drain_group(s, ue_h, bufs, sem)
        acc = _compute_group(s, acc, bufs, d_v, lane)

        @pl.when(s < GROUPS - DEPTH)
        def _():
            _issue_group(s + DEPTH, idx_u, idx_p, idx_n, ue_h, ie_h, bufs, sem)

        return acc

    acc = lax.fori_loop(0, GROUPS, step, jnp.zeros((16,), jnp.float32))
    acc_v[...] = acc

    pltpu.sync_copy(d_v, d_out.at[pl.ds(base, BPW)])
    pltpu.sync_copy(acc_v, acc_out.at[wid])


@functools.partial(
    pl.kernel,
    mesh=plsc.VectorSubcoreMesh(core_axis_name="c", subcore_axis_name="s"),
    compiler_params=pltpu.CompilerParams(
        needs_layout_passes=False, use_tc_tiling_on_sc=True),
    out_type=[
        jax.ShapeDtypeStruct((BATCH,), jnp.float32),
        jax.ShapeDtypeStruct((NW, EMBED), jnp.float32),
    ],
    scratch_types=[
        pltpu.VMEM((BPW,), jnp.int32),
        pltpu.VMEM((BPW,), jnp.int32),
        pltpu.VMEM((BPW,), jnp.int32),
        pltpu.VMEM((DEPTH, 16, EMBED), jnp.float32),
        pltpu.VMEM((DEPTH, 16, EMBED), jnp.float32),
        pltpu.VMEM((DEPTH, 16, EMBED), jnp.float32),
        pltpu.VMEM((BPW,), jnp.float32),
        pltpu.VMEM((EMBED,), jnp.float32),
        pltpu.SemaphoreType.DMA,
    ],
)
def _sc_kernel(user_h, pos_h, neg_h, ue_h, ie_h, d_out, acc_out,
               idx_u, idx_p, idx_n, bu_v, bp_v, bn_v, d_v, acc_v, sem):
    _sc_body(user_h, pos_h, neg_h, ue_h, ie_h, d_out, acc_out,
             idx_u, idx_p, idx_n, bu_v, bp_v, bn_v, d_v, acc_v, sem)


def _tc_body(d_ref, acc_ref, bpr_ref, reg_ref):
    x = d_ref[...]
    s = jnp.log(jax.nn.sigmoid(x))
    bpr_ref[0, 0] = -jnp.sum(s) / jnp.float32(BATCH)
    reg_ref[0, 0] = jnp.float32(REG_SCALE) * jnp.sum(acc_ref[...])


_tc_finish = pl.pallas_call(
    _tc_body,
    out_shape=[
        jax.ShapeDtypeStruct((1, 1), jnp.float32),
        jax.ShapeDtypeStruct((1, 1), jnp.float32),
    ],
    in_specs=[
        pl.BlockSpec(memory_space=pltpu.VMEM),
        pl.BlockSpec(memory_space=pltpu.VMEM),
    ],
    out_specs=[
        pl.BlockSpec(memory_space=pltpu.SMEM),
        pl.BlockSpec(memory_space=pltpu.SMEM),
    ],
)


def kernel(user, pos, neg, user_embedding, item_embedding):
    user = user.astype(jnp.int32)
    pos = pos.astype(jnp.int32)
    neg = neg.astype(jnp.int32)
    d, acc = _sc_kernel(user, pos, neg, user_embedding, item_embedding)
    bpr, reg = _tc_finish(d.reshape(128, 128), acc)
    return (bpr[0, 0], reg[0, 0])
